# baseline (speedup 1.0000x reference)
import jax
import jax.numpy as jnp
from jax import lax
from jax.experimental import pallas as pl
from jax.experimental.pallas import tpu as pltpu
from jax.experimental.pallas import tpu_sc as plsc

N = 50000
E = 800000
B = 16
D_IN = 20
D_HID = 128
D_OUT = 64

NC, NS = 2, 16
NW = NC * NS
CHUNK = 128
NCHUNK = E // CHUNK
NITER = -(-NCHUNK // NW)
NPAD = 51200
RPT = NPAD // NS
DPAD = 32
BLK = 2048
NB = NPAD // BLK

_MESH = plsc.VectorSubcoreMesh(
    core_axis_name="c", subcore_axis_name="s", num_cores=NC, num_subcores=NS
)


def _deg_body(edges_hbm, z1_hbm, degp_hbm,
              src0, src1, dst0, dst1, isem0, isem1, osem0, osem1, xsem0, xsem1,
              ones_v, dego_sh, degi_sh):
    srcs, dsts, isems = (src0, src1), (dst0, dst1), (isem0, isem1)
    osems, xsems = (osem0, osem1), (xsem0, xsem1)
    cid = lax.axis_index("c")
    sid = lax.axis_index("s")
    wid = cid * NS + sid
    stripe = pl.ds(sid * RPT, RPT)
    pltpu.sync_copy(z1_hbm, dego_sh.at[stripe])
    pltpu.sync_copy(z1_hbm, degi_sh.at[stripe])
    for i in range(8):
        ones_v[pl.ds(i * 16, 16)] = jnp.ones((16,), jnp.float32)
    plsc.subcore_barrier()

    def issue_idx(j, b):
        c = wid + j * NW

        @pl.when(c < NCHUNK)
        def _():
            pltpu.async_copy(edges_hbm.at[0].at[c], srcs[b], isems[b])
            pltpu.async_copy(edges_hbm.at[1].at[c], dsts[b], isems[b])

    def scatter(j, b):
        c = wid + j * NW

        @pl.when(c < NCHUNK)
        def _():
            pltpu.make_async_copy(edges_hbm.at[0].at[c], srcs[b], isems[b]).wait()
            pltpu.make_async_copy(edges_hbm.at[1].at[c], dsts[b], isems[b]).wait()
            pltpu.async_copy(ones_v, dego_sh.at[srcs[b]], osems[b], add=True)
            pltpu.async_copy(ones_v, degi_sh.at[dsts[b]], xsems[b], add=True)

    def drain(j, b):
        c = wid + j * NW

        @pl.when(c < NCHUNK)
        def _():
            pltpu.make_async_copy(ones_v, dego_sh.at[srcs[b]], osems[b]).wait()
            pltpu.make_async_copy(ones_v, degi_sh.at[dsts[b]], xsems[b]).wait()

    issue_idx(0, 0)
    issue_idx(1, 1)
    scatter(0, 0)

    def group(g, carry):
        j0 = g * 2
        for b in range(2):
            j = j0 + b
            scatter(j + 1, (b + 1) % 2)
            drain(j, b)
            issue_idx(j + 2, b)
        return carry

    lax.fori_loop(0, NITER // 2, group, 0)
    plsc.subcore_barrier()
    pltpu.sync_copy(dego_sh.at[stripe], degp_hbm.at[cid].at[0].at[stripe])
    pltpu.sync_copy(degi_sh.at[stripe], degp_hbm.at[cid].at[1].at[stripe])


_deg_call = pl.kernel(
    _deg_body,
    out_type=jax.ShapeDtypeStruct((NC, 2, NPAD), jnp.float32),
    mesh=_MESH,
    scratch_types=(
        [pltpu.VMEM((CHUNK,), jnp.int32)] * 4
        + [pltpu.SemaphoreType.DMA] * 6
        + [pltpu.VMEM((CHUNK,), jnp.float32)]
        + [pltpu.VMEM_SHARED((NPAD,), jnp.float32)] * 2
    ),
)


def _make_agg(P):
    def body(*args):
        tables_hbm = args[0]
        edges_hbm, zrows_hbm, out_hbm = args[1], args[2], args[3]
        K = 6
        srcs = args[4:4 + K]
        dsts = args[4 + K:4 + 2 * K]
        rows = args[4 + 2 * K:4 + 3 * K]
        isems = args[4 + 3 * K:4 + 4 * K]
        gsems = args[4 + 4 * K:4 + 5 * K]
        ssems = args[4 + 5 * K:4 + 6 * K]
        agg_sh = args[4 + 6 * K]
        cid = lax.axis_index("c")
        sid = lax.axis_index("s")
        wid = cid * NS + sid
        stripe = pl.ds(sid * RPT, RPT)

        def issue_idx(j, b):
            c = wid + j * NW

            @pl.when(c < NCHUNK)
            def _():
                pltpu.async_copy(edges_hbm.at[0].at[c], srcs[b], isems[b])
                pltpu.async_copy(edges_hbm.at[1].at[c], dsts[b], isems[b])

        def issue_gather(j, b, table):
            c = wid + j * NW

            @pl.when(c < NCHUNK)
            def _():
                pltpu.make_async_copy(edges_hbm.at[0].at[c], srcs[b], isems[b]).wait()
                pltpu.make_async_copy(edges_hbm.at[1].at[c], dsts[b], isems[b]).wait()
                pltpu.async_copy(table.at[srcs[b]], rows[b], gsems[b])

        def wait_gather(j, b, table):
            c = wid + j * NW

            @pl.when(c < NCHUNK)
            def _():
                pltpu.make_async_copy(table.at[srcs[b]], rows[b], gsems[b]).wait()

        def issue_scatter(j, b):
            c = wid + j * NW

            @pl.when(c < NCHUNK)
            def _():
                pltpu.async_copy(rows[b], agg_sh.at[dsts[b]], ssems[b], add=True)

        def wait_scatter(j, b):
            c = wid + j * NW

            @pl.when(c < NCHUNK)
            def _():
                pltpu.make_async_copy(rows[b], agg_sh.at[dsts[b]], ssems[b]).wait()

        ngroup = -(-NITER // K)
        for p in range(P):
            table = tables_hbm.at[p]
            pltpu.sync_copy(zrows_hbm, agg_sh.at[stripe])
            plsc.subcore_barrier()
            for b in range(K):
                issue_idx(b, b)

            def group(g, carry, table=table):
                j0 = g * K
                for b in range(K):
                    issue_gather(j0 + b, b, table)
                for b in range(K):
                    wait_gather(j0 + b, b, table)
                for b in range(K):
                    issue_scatter(j0 + b, b)
                for b in range(K):
                    wait_scatter(j0 + b, b)
                    issue_idx(j0 + K + b, b)
                return carry

            lax.fori_loop(0, ngroup, group, 0)
            plsc.subcore_barrier()
            pltpu.sync_copy(agg_sh.at[stripe], out_hbm.at[cid].at[p].at[stripe])
            if p + 1 < P:
                plsc.subcore_barrier()

    return pl.kernel(
        body,
        out_type=jax.ShapeDtypeStruct((NC, P, NPAD, DPAD), jnp.float32),
        mesh=_MESH,
        compiler_params=pltpu.CompilerParams(use_tc_tiling_on_sc=False),
        scratch_types=(
            [pltpu.VMEM((CHUNK,), jnp.int32)] * 12
            + [pltpu.VMEM((CHUNK, DPAD), jnp.float32)] * 6
            + [pltpu.SemaphoreType.DMA] * 18
            + [pltpu.VMEM_SHARED((NPAD, DPAD), jnp.float32)]
        ),
    )


_agg1_call = _make_agg(1)
_agg2_call = _make_agg(2)


def _mm1_body(h_ref, degp_ref, feat1_ref):
    dego = jnp.maximum(degp_ref[0, 0, :] + degp_ref[1, 0, :], 1.0)
    f = h_ref[...] * lax.rsqrt(dego)[:, None]
    feat1_ref[...] = jnp.concatenate(
        [f, jnp.zeros((BLK, DPAD - D_IN), jnp.float32)], axis=1
    )


def _mm2_body(aggp_ref, degp_ref, w1_ref, b1_ref, w2_ref, feat2_ref):
    a = aggp_ref[0, 0] + aggp_ref[1, 0]
    dego = jnp.maximum(degp_ref[0, 0, :] + degp_ref[1, 0, :], 1.0)
    degi = jnp.maximum(degp_ref[0, 1, :] + degp_ref[1, 1, :], 1.0)
    rst = jnp.dot(a, w1_ref[...], preferred_element_type=jnp.float32)
    out1 = jnp.maximum(rst * lax.rsqrt(degi)[:, None] + b1_ref[...], 0.0)
    f2 = jnp.dot(out1 * lax.rsqrt(dego)[:, None], w2_ref[...],
                 preferred_element_type=jnp.float32)
    feat2_ref[0] = f2[:, :DPAD]
    feat2_ref[1] = f2[:, DPAD:]


def _fin_body(aggp_ref, degp_ref, b2_ref, out_ref):
    a = jnp.concatenate(
        [aggp_ref[0, 0] + aggp_ref[1, 0], aggp_ref[0, 1] + aggp_ref[1, 1]], axis=1
    )
    degi = jnp.maximum(degp_ref[0, 1, :] + degp_ref[1, 1, :], 1.0)
    out_ref[...] = jnp.maximum(a * lax.rsqrt(degi)[:, None] + b2_ref[...], 0.0)


def kernel(h, edge_index, batch_num_nodes, W1, b1, W2, b2):
    del batch_num_nodes
    edges_r = edge_index.reshape(2, NCHUNK, CHUNK)
    hp = jnp.concatenate([h, jnp.zeros((NPAD - N, D_IN), h.dtype)], axis=0)
    z1 = jnp.zeros((RPT,), jnp.float32)
    zrows = jnp.zeros((RPT, DPAD), jnp.float32)
    w1p = jnp.concatenate([W1, jnp.zeros((DPAD - D_IN, D_HID), W1.dtype)], axis=0)

    degp = _deg_call(edges_r, z1)

    feat1 = pl.pallas_call(
        _mm1_body,
        grid=(NB,),
        in_specs=[
            pl.BlockSpec((BLK, D_IN), lambda i: (i, 0)),
            pl.BlockSpec((NC, 2, BLK), lambda i: (0, 0, i)),
        ],
        out_specs=pl.BlockSpec((BLK, DPAD), lambda i: (i, 0)),
        out_shape=jax.ShapeDtypeStruct((NPAD, DPAD), jnp.float32),
    )(hp, degp)

    agg1p = _agg1_call(feat1[None], edges_r, zrows)

    feat2 = pl.pallas_call(
        _mm2_body,
        grid=(NB,),
        in_specs=[
            pl.BlockSpec((NC, 1, BLK, DPAD), lambda i: (0, 0, i, 0)),
            pl.BlockSpec((NC, 2, BLK), lambda i: (0, 0, i)),
            pl.BlockSpec((DPAD, D_HID), lambda i: (0, 0)),
            pl.BlockSpec((1, D_HID), lambda i: (0, 0)),
            pl.BlockSpec((D_HID, D_OUT), lambda i: (0, 0)),
        ],
        out_specs=pl.BlockSpec((2, BLK, DPAD), lambda i: (0, i, 0)),
        out_shape=jax.ShapeDtypeStruct((2, NPAD, DPAD), jnp.float32),
    )(agg1p, degp, w1p, b1[None, :], W2)

    agg2p = _agg2_call(feat2, edges_r, zrows)

    out = pl.pallas_call(
        _fin_body,
        grid=(NB,),
        in_specs=[
            pl.BlockSpec((NC, 2, BLK, DPAD), lambda i: (0, 0, i, 0)),
            pl.BlockSpec((NC, 2, BLK), lambda i: (0, 0, i)),
            pl.BlockSpec((1, D_OUT), lambda i: (0, 0)),
        ],
        out_specs=pl.BlockSpec((BLK, D_OUT), lambda i: (i, 0)),
        out_shape=jax.ShapeDtypeStruct((NPAD, D_OUT), jnp.float32),
    )(agg2p, degp, b2[None, :])

    return out[:N].reshape(B, N // B, D_OUT)


# speedup vs baseline: 1.0406x; 1.0406x over previous
import jax
import jax.numpy as jnp
from jax import lax
from jax.experimental import pallas as pl
from jax.experimental.pallas import tpu as pltpu
from jax.experimental.pallas import tpu_sc as plsc

N = 50000
E = 800000
B = 16
D_IN = 20
D_HID = 128
D_OUT = 64

NC, NS = 2, 16
NW = NC * NS
CHUNK = 128
NCHUNK = E // CHUNK
NITER = -(-NCHUNK // NW)
NPAD = 51200
RPT = NPAD // NS
DPAD = 32
BLK = 2048
NB = NPAD // BLK

_MESH = plsc.VectorSubcoreMesh(
    core_axis_name="c", subcore_axis_name="s", num_cores=NC, num_subcores=NS
)


def _deg_body(edges_hbm, z1_hbm, degp_hbm,
              src0, src1, dst0, dst1, isem0, isem1, ones_v, dego_sh, degi_sh):
    srcs, dsts, isems = (src0, src1), (dst0, dst1), (isem0, isem1)
    cid = lax.axis_index("c")
    sid = lax.axis_index("s")
    wid = cid * NS + sid
    stripe = pl.ds(sid * RPT, RPT)
    pltpu.sync_copy(z1_hbm, dego_sh.at[stripe])
    pltpu.sync_copy(z1_hbm, degi_sh.at[stripe])
    for i in range(8):
        ones_v[pl.ds(i * 16, 16)] = jnp.ones((16,), jnp.float32)
    plsc.subcore_barrier()

    def issue_idx(j, b):
        c = wid + j * NW

        @pl.when(c < NCHUNK)
        def _():
            pltpu.async_copy(edges_hbm.at[0].at[c], srcs[b], isems[b])
            pltpu.async_copy(edges_hbm.at[1].at[c], dsts[b], isems[b])

    def scatter(j, b):
        c = wid + j * NW

        @pl.when(c < NCHUNK)
        def _():
            pltpu.make_async_copy(edges_hbm.at[0].at[c], srcs[b], isems[b]).wait()
            pltpu.make_async_copy(edges_hbm.at[1].at[c], dsts[b], isems[b]).wait()
            pltpu.sync_copy(ones_v, dego_sh.at[srcs[b]], add=True)
            pltpu.sync_copy(ones_v, degi_sh.at[dsts[b]], add=True)

    issue_idx(0, 0)

    def group(g, carry):
        j0 = g * 2
        for b in range(2):
            j = j0 + b
            issue_idx(j + 1, (b + 1) % 2)
            scatter(j, b)
        return carry

    lax.fori_loop(0, NITER // 2, group, 0)
    plsc.subcore_barrier()
    pltpu.sync_copy(dego_sh.at[stripe], degp_hbm.at[cid].at[0].at[stripe])
    pltpu.sync_copy(degi_sh.at[stripe], degp_hbm.at[cid].at[1].at[stripe])


_deg_call = pl.kernel(
    _deg_body,
    out_type=jax.ShapeDtypeStruct((NC, 2, NPAD), jnp.float32),
    mesh=_MESH,
    scratch_types=(
        [pltpu.VMEM((CHUNK,), jnp.int32)] * 4
        + [pltpu.SemaphoreType.DMA] * 2
        + [pltpu.VMEM((CHUNK,), jnp.float32)]
        + [pltpu.VMEM_SHARED((NPAD,), jnp.float32)] * 2
    ),
)


def _make_agg(P):
    def body(*args):
        tables_hbm = args[0]
        edges_hbm, zrows_hbm, out_hbm = args[1], args[2], args[3]
        K = 6
        srcs = args[4:4 + K]
        dsts = args[4 + K:4 + 2 * K]
        rows = args[4 + 2 * K:4 + 3 * K]
        isems = args[4 + 3 * K:4 + 4 * K]
        gsems = args[4 + 4 * K:4 + 5 * K]
        ssems = args[4 + 5 * K:4 + 6 * K]
        agg_sh = args[4 + 6 * K]
        cid = lax.axis_index("c")
        sid = lax.axis_index("s")
        wid = cid * NS + sid
        stripe = pl.ds(sid * RPT, RPT)

        def issue_idx(j, b):
            c = wid + j * NW

            @pl.when(c < NCHUNK)
            def _():
                pltpu.async_copy(edges_hbm.at[0].at[c], srcs[b], isems[b])
                pltpu.async_copy(edges_hbm.at[1].at[c], dsts[b], isems[b])

        def issue_gather(j, b, table):
            c = wid + j * NW

            @pl.when(c < NCHUNK)
            def _():
                pltpu.make_async_copy(edges_hbm.at[0].at[c], srcs[b], isems[b]).wait()
                pltpu.make_async_copy(edges_hbm.at[1].at[c], dsts[b], isems[b]).wait()
                pltpu.async_copy(table.at[srcs[b]], rows[b], gsems[b])

        def wait_gather(j, b, table):
            c = wid + j * NW

            @pl.when(c < NCHUNK)
            def _():
                pltpu.make_async_copy(table.at[srcs[b]], rows[b], gsems[b]).wait()

        def issue_scatter(j, b):
            c = wid + j * NW

            @pl.when(c < NCHUNK)
            def _():
                pltpu.async_copy(rows[b], agg_sh.at[dsts[b]], ssems[b], add=True)

        def wait_scatter(j, b):
            c = wid + j * NW

            @pl.when(c < NCHUNK)
            def _():
                pltpu.make_async_copy(rows[b], agg_sh.at[dsts[b]], ssems[b]).wait()

        ngroup = -(-NITER // K)
        for p in range(P):
            table = tables_hbm.at[p]
            pltpu.sync_copy(zrows_hbm, agg_sh.at[stripe])
            plsc.subcore_barrier()
            for b in range(K):
                issue_idx(b, b)

            def group(g, carry, table=table):
                j0 = g * K
                for b in range(K):
                    issue_gather(j0 + b, b, table)
                for b in range(K):
                    wait_gather(j0 + b, b, table)
                for b in range(K):
                    issue_scatter(j0 + b, b)
                for b in range(K):
                    wait_scatter(j0 + b, b)
                    issue_idx(j0 + K + b, b)
                return carry

            lax.fori_loop(0, ngroup, group, 0)
            plsc.subcore_barrier()
            pltpu.sync_copy(agg_sh.at[stripe], out_hbm.at[cid].at[p].at[stripe])
            if p + 1 < P:
                plsc.subcore_barrier()

    return pl.kernel(
        body,
        out_type=jax.ShapeDtypeStruct((NC, P, NPAD, DPAD), jnp.float32),
        mesh=_MESH,
        compiler_params=pltpu.CompilerParams(use_tc_tiling_on_sc=False),
        scratch_types=(
            [pltpu.VMEM((CHUNK,), jnp.int32)] * 12
            + [pltpu.VMEM((CHUNK, DPAD), jnp.float32)] * 6
            + [pltpu.SemaphoreType.DMA] * 18
            + [pltpu.VMEM_SHARED((NPAD, DPAD), jnp.float32)]
        ),
    )


_agg1_call = _make_agg(1)
_agg2_call = _make_agg(2)


def _mm1_body(h_ref, degp_ref, feat1_ref):
    dego = jnp.maximum(degp_ref[0, 0, :] + degp_ref[1, 0, :], 1.0)
    f = h_ref[...] * lax.rsqrt(dego)[:, None]
    feat1_ref[...] = jnp.concatenate(
        [f, jnp.zeros((BLK, DPAD - D_IN), jnp.float32)], axis=1
    )


def _mm2_body(aggp_ref, degp_ref, w1_ref, b1_ref, w2_ref, feat2_ref):
    a = aggp_ref[0, 0] + aggp_ref[1, 0]
    dego = jnp.maximum(degp_ref[0, 0, :] + degp_ref[1, 0, :], 1.0)
    degi = jnp.maximum(degp_ref[0, 1, :] + degp_ref[1, 1, :], 1.0)
    rst = jnp.dot(a, w1_ref[...], preferred_element_type=jnp.float32)
    out1 = jnp.maximum(rst * lax.rsqrt(degi)[:, None] + b1_ref[...], 0.0)
    f2 = jnp.dot(out1 * lax.rsqrt(dego)[:, None], w2_ref[...],
                 preferred_element_type=jnp.float32)
    feat2_ref[0] = f2[:, :DPAD]
    feat2_ref[1] = f2[:, DPAD:]


def _fin_body(aggp_ref, degp_ref, b2_ref, out_ref):
    a = jnp.concatenate(
        [aggp_ref[0, 0] + aggp_ref[1, 0], aggp_ref[0, 1] + aggp_ref[1, 1]], axis=1
    )
    degi = jnp.maximum(degp_ref[0, 1, :] + degp_ref[1, 1, :], 1.0)
    out_ref[...] = jnp.maximum(a * lax.rsqrt(degi)[:, None] + b2_ref[...], 0.0)


def kernel(h, edge_index, batch_num_nodes, W1, b1, W2, b2):
    del batch_num_nodes
    edges_r = edge_index.reshape(2, NCHUNK, CHUNK)
    hp = jnp.concatenate([h, jnp.zeros((NPAD - N, D_IN), h.dtype)], axis=0)
    z1 = jnp.zeros((RPT,), jnp.float32)
    zrows = jnp.zeros((RPT, DPAD), jnp.float32)
    w1p = jnp.concatenate([W1, jnp.zeros((DPAD - D_IN, D_HID), W1.dtype)], axis=0)

    degp = _deg_call(edges_r, z1)

    feat1 = pl.pallas_call(
        _mm1_body,
        grid=(NB,),
        in_specs=[
            pl.BlockSpec((BLK, D_IN), lambda i: (i, 0)),
            pl.BlockSpec((NC, 2, BLK), lambda i: (0, 0, i)),
        ],
        out_specs=pl.BlockSpec((BLK, DPAD), lambda i: (i, 0)),
        out_shape=jax.ShapeDtypeStruct((NPAD, DPAD), jnp.float32),
    )(hp, degp)

    agg1p = _agg1_call(feat1[None], edges_r, zrows)

    feat2 = pl.pallas_call(
        _mm2_body,
        grid=(NB,),
        in_specs=[
            pl.BlockSpec((NC, 1, BLK, DPAD), lambda i: (0, 0, i, 0)),
            pl.BlockSpec((NC, 2, BLK), lambda i: (0, 0, i)),
            pl.BlockSpec((DPAD, D_HID), lambda i: (0, 0)),
            pl.BlockSpec((1, D_HID), lambda i: (0, 0)),
            pl.BlockSpec((D_HID, D_OUT), lambda i: (0, 0)),
        ],
        out_specs=pl.BlockSpec((2, BLK, DPAD), lambda i: (0, i, 0)),
        out_shape=jax.ShapeDtypeStruct((2, NPAD, DPAD), jnp.float32),
    )(agg1p, degp, w1p, b1[None, :], W2)

    agg2p = _agg2_call(feat2, edges_r, zrows)

    out = pl.pallas_call(
        _fin_body,
        grid=(NB,),
        in_specs=[
            pl.BlockSpec((NC, 2, BLK, DPAD), lambda i: (0, 0, i, 0)),
            pl.BlockSpec((NC, 2, BLK), lambda i: (0, 0, i)),
            pl.BlockSpec((1, D_OUT), lambda i: (0, 0)),
        ],
        out_specs=pl.BlockSpec((BLK, D_OUT), lambda i: (i, 0)),
        out_shape=jax.ShapeDtypeStruct((NPAD, D_OUT), jnp.float32),
    )(agg2p, degp, b2[None, :])

    return out[:N].reshape(B, N // B, D_OUT)
